# baseline (speedup 1.0000x reference)
import functools
import math

import jax
import jax.numpy as jnp
from jax import lax
from jax.experimental import pallas as pl
from jax.experimental.pallas import tpu as pltpu
from jax.experimental.pallas import tpu_sc as plsc

EMB = 64
SCALE = math.sqrt(EMB)
LANES = 16
BLK = 128


@functools.lru_cache(maxsize=None)
def _make_kernel(S, B, NC, NS):
    NW = NC * NS
    n_blocks_total = (S * B) // BLK
    n_blocks = n_blocks_total // NW
    blocks_per_s = B // BLK
    mesh = plsc.VectorSubcoreMesh(core_axis_name="c", subcore_axis_name="s")

    @functools.partial(
        pl.kernel,
        mesh=mesh,
        compiler_params=pltpu.CompilerParams(
            use_tc_tiling_on_sc=True, needs_layout_passes=False
        ),
        out_type=jax.ShapeDtypeStruct((S, EMB, B), jnp.float32),
        scratch_types=[
            pltpu.VMEM((BLK,), jnp.int32),
            pltpu.VMEM((BLK,), jnp.int32),
            pltpu.VMEM((BLK,), jnp.int32),
            pltpu.VMEM((BLK, 2 * EMB), jnp.float32),
            pltpu.VMEM((EMB, BLK), jnp.float32),
            pltpu.SemaphoreType.DMA,
        ],
    )
    def k(tokens_hbm, table_hbm, out_hbm, tkb, qidx, half, gbuf, obuf, sem):
        wid = lax.axis_index("s") * NC + lax.axis_index("c")
        base = wid * n_blocks

        def block_body(j, carry):
            bid = base + j
            s = bid // blocks_per_s
            c = bid % blocks_per_s

            pltpu.sync_copy(tokens_hbm.at[s, pl.ds(c * BLK, BLK)], tkb)
            for kk in range(BLK // LANES):
                sl = pl.ds(kk * LANES, LANES)
                tv = tkb[sl]
                qidx[sl] = lax.shift_right_logical(tv, 1)
                half[sl] = lax.shift_left(jnp.bitwise_and(tv, 1), 6)
            pltpu.async_copy(table_hbm.at[qidx], gbuf, sem).wait()

            rows = [
                lax.iota(jnp.int32, LANES) + kk * LANES
                for kk in range(BLK // LANES)
            ]
            halves = [half[pl.ds(kk * LANES, LANES)] for kk in range(BLK // LANES)]

            def d_body(d, c2):
                for kk in range(BLK // LANES):
                    cols = halves[kk] + d
                    vals = plsc.load_gather(gbuf, [rows[kk], cols])
                    obuf[d, pl.ds(kk * LANES, LANES)] = vals * SCALE
                return c2

            lax.fori_loop(0, EMB, d_body, 0)
            pltpu.sync_copy(obuf, out_hbm.at[s, :, pl.ds(c * BLK, BLK)])
            return carry

        lax.fori_loop(0, n_blocks, block_body, 0)

    return k


def kernel(tokens, table):
    B, S = tokens.shape
    info = plsc.get_sparse_core_info()
    k = _make_kernel(S, B, info.num_cores, info.num_subcores)
    tokens_t = tokens.T.astype(jnp.int32)
    table2 = table.reshape(table.shape[0] // 2, 2 * table.shape[1])
    out = k(tokens_t, table2)
    return jnp.transpose(out, (2, 0, 1))


# speedup vs baseline: 1.2252x; 1.2252x over previous
import functools
import math

import jax
import jax.numpy as jnp
from jax import lax
from jax.experimental import pallas as pl
from jax.experimental.pallas import tpu as pltpu
from jax.experimental.pallas import tpu_sc as plsc

EMB = 64
SCALE = math.sqrt(EMB)
LANES = 16
BLK = 128
NBUF = 2


@functools.lru_cache(maxsize=None)
def _make_kernel(S, B, NC, NS):
    NW = NC * NS
    n_blocks_total = (S * B) // BLK
    n_blocks = n_blocks_total // NW
    blocks_per_s = B // BLK
    mesh = plsc.VectorSubcoreMesh(core_axis_name="c", subcore_axis_name="s")

    @functools.partial(
        pl.kernel,
        mesh=mesh,
        compiler_params=pltpu.CompilerParams(
            use_tc_tiling_on_sc=True, needs_layout_passes=False
        ),
        out_type=jax.ShapeDtypeStruct((S, EMB, B), jnp.float32),
        scratch_types=[
            pltpu.VMEM((NBUF, BLK), jnp.int32),
            pltpu.VMEM((NBUF, BLK), jnp.int32),
            pltpu.VMEM((NBUF, BLK), jnp.int32),
            pltpu.VMEM((NBUF, BLK, 2 * EMB), jnp.float32),
            pltpu.VMEM((NBUF, EMB, BLK), jnp.float32),
            pltpu.SemaphoreType.DMA((NBUF,)),
            pltpu.SemaphoreType.DMA((NBUF,)),
            pltpu.SemaphoreType.DMA((NBUF,)),
        ],
    )
    def k(tokens_hbm, table_hbm, out_hbm, tbuf, qbuf, hbuf, gbuf, obuf,
          tsem, gsem, osem):
        wid = lax.axis_index("s") * NC + lax.axis_index("c")
        base = wid * n_blocks

        def tok_slice(j):
            bid = base + j
            s = bid // blocks_per_s
            c = bid % blocks_per_s
            return tokens_hbm.at[s, pl.ds(c * BLK, BLK)]

        def start_tok(j, b):
            pltpu.async_copy(tok_slice(j), tbuf.at[b], tsem.at[b])

        def wait_tok(j, b):
            pltpu.make_async_copy(tok_slice(j), tbuf.at[b], tsem.at[b]).wait()

        def split(b):
            for kk in range(BLK // LANES):
                sl = pl.ds(kk * LANES, LANES)
                tv = tbuf[b, sl]
                hbuf[b, sl] = lax.shift_left(jnp.bitwise_and(tv, 1), 6)
                qbuf[b, sl] = lax.shift_right_logical(tv, 1)

        def start_gather(b):
            pltpu.async_copy(table_hbm.at[qbuf.at[b]], gbuf.at[b], gsem.at[b])

        def wait_gather(b):
            pltpu.make_async_copy(
                table_hbm.at[qbuf.at[b]], gbuf.at[b], gsem.at[b]
            ).wait()

        def out_slice(j):
            bid = base + j
            s = bid // blocks_per_s
            c = bid % blocks_per_s
            return out_hbm.at[s, :, pl.ds(c * BLK, BLK)]

        def start_write(j, b):
            pltpu.async_copy(obuf.at[b], out_slice(j), osem.at[b])

        def wait_write(j, b):
            pltpu.make_async_copy(obuf.at[b], out_slice(j), osem.at[b]).wait()

        rows = [
            lax.iota(jnp.int32, LANES) + kk * LANES for kk in range(BLK // LANES)
        ]

        start_tok(0, 0)
        wait_tok(0, 0)
        split(0)
        start_gather(0)
        start_tok(1, 1)

        def block_body(j2, carry):
            for b in range(NBUF):
                jj = j2 * NBUF + b
                nb = (b + 1) % NBUF

                @pl.when(jj + 1 < n_blocks)
                def _():
                    wait_tok(jj + 1, nb)
                    split(nb)
                    start_gather(nb)

                @pl.when(jj + 2 < n_blocks)
                def _():
                    start_tok(jj + 2, b)

                wait_gather(b)

                @pl.when(jj >= NBUF)
                def _():
                    wait_write(jj - NBUF, b)

                halves = [
                    hbuf[b, pl.ds(kk * LANES, LANES)]
                    for kk in range(BLK // LANES)
                ]

                def d_body(d, c2):
                    for kk in range(BLK // LANES):
                        cols = halves[kk] + d
                        vals = plsc.load_gather(gbuf.at[b], [rows[kk], cols])
                        obuf[b, d, pl.ds(kk * LANES, LANES)] = vals * SCALE
                    return c2

                lax.fori_loop(0, EMB, d_body, 0)
                start_write(jj, b)
            return carry

        lax.fori_loop(0, n_blocks // NBUF, block_body, 0)
        for b in range(NBUF):
            wait_write(n_blocks - NBUF + b, b)

    return k


def kernel(tokens, table):
    B, S = tokens.shape
    info = plsc.get_sparse_core_info()
    k = _make_kernel(S, B, info.num_cores, info.num_subcores)
    tokens_t = tokens.T.astype(jnp.int32)
    table2 = table.reshape(table.shape[0] // 2, 2 * table.shape[1])
    out = k(tokens_t, table2)
    return jnp.transpose(out, (2, 0, 1))
